# baseline (speedup 1.0000x reference)
import functools

import jax
import jax.numpy as jnp
from jax import lax
from jax.experimental import pallas as pl
from jax.experimental.pallas import tpu as pltpu
from jax.experimental.pallas import tpu_sc as plsc

_V = 1024
_F = 2048
_T = 4096
_UP = 3
_K = 5
_STD = 0.05
_Q = _T * _UP + _F
_BQ = 256
_G = 512
_NO = _T // _G

_f32 = jnp.float32
_i32 = jnp.int32
_IMAX = jnp.iinfo(jnp.int32).max


_B_GATHER = 3 * _F
_DPAD = 128


@functools.lru_cache(maxsize=1)
def _make_sc_gather():
    info = plsc.get_sparse_core_info()
    nc, ns = info.num_cores, info.num_subcores
    nw = nc * ns
    b_per_w = _B_GATHER // nw
    chunk = 96
    nchunk = b_per_w // chunk
    assert b_per_w % chunk == 0
    mesh = plsc.VectorSubcoreMesh(core_axis_name="c", subcore_axis_name="s")

    @functools.partial(
        pl.kernel,
        mesh=mesh,
        out_type=jax.ShapeDtypeStruct((_B_GATHER, _DPAD), _f32),
        scratch_types=[
            pltpu.VMEM((chunk,), _i32),
            pltpu.VMEM((chunk, _DPAD), _f32),
            pltpu.SemaphoreType.DMA,
        ],
    )
    def k(table_hbm, idx_hbm, out_hbm, idx_v, rows_v, sem):
        wid = lax.axis_index("s") * nc + lax.axis_index("c")
        base = wid * b_per_w
        for j in range(nchunk):
            off = base + j * chunk
            pltpu.sync_copy(idx_hbm.at[pl.ds(off, chunk)], idx_v)
            pltpu.async_copy(table_hbm.at[idx_v], rows_v, sem).wait()
            pltpu.sync_copy(rows_v, out_hbm.at[pl.ds(off, chunk)])

    return k


def _prep_tab_kernel(rt_ref, ftab_ref):
    a = rt_ref[0:3, 0:_F]
    b = rt_ref[0:3, _F : 2 * _F]
    c = rt_ref[0:3, 2 * _F : 3 * _F]
    ab = b - a
    ac = c - a
    ftab_ref[0:3, :] = a
    ftab_ref[3:6, :] = ab
    ftab_ref[6:9, :] = ac
    aa = jnp.sum(ab * ab, axis=0, keepdims=True)
    e = jnp.sum(ab * ac, axis=0, keepdims=True)
    cc = jnp.sum(ac * ac, axis=0, keepdims=True)
    gce = cc - e
    gae = aa - e
    ftab_ref[9:10, :] = jnp.sum(ab * a, axis=0, keepdims=True)
    ftab_ref[10:11, :] = jnp.sum(ac * a, axis=0, keepdims=True)
    ftab_ref[11:12, :] = aa
    ftab_ref[12:13, :] = e
    ftab_ref[13:14, :] = cc
    ftab_ref[14:15, :] = gce
    ftab_ref[15:16, :] = gae
    ftab_ref[16:17, :] = gae * cc + e * gce
    ftab_ref[17:24, :] = jnp.zeros((7, _F), _f32)


def _main_kernel(qp_ref, ftab_ref, tgtt_ref, tgq_ref, out_ref):
    i = pl.program_id(0)

    p = qp_ref[:, :]
    px = p[:, 0:1]
    py = p[:, 1:2]
    pz = p[:, 2:3]

    abx = ftab_ref[3:4, :]
    aby = ftab_ref[4:5, :]
    abz = ftab_ref[5:6, :]
    acx = ftab_ref[6:7, :]
    acy = ftab_ref[7:8, :]
    acz = ftab_ref[8:9, :]
    d1 = ((abx * px + aby * py) + abz * pz) - ftab_ref[9:10, :]
    d2_ = ((acx * px + acy * py) + acz * pz) - ftab_ref[10:11, :]
    aa = ftab_ref[11:12, :]
    e = ftab_ref[12:13, :]
    cc = ftab_ref[13:14, :]
    gce = ftab_ref[14:15, :]
    gae = ftab_ref[15:16, :]
    det = ftab_ref[16:17, :]
    d3 = d1 - aa
    d4 = d2_ - e
    d5 = d1 - e
    d6 = d2_ - cc

    d21 = d2_ - d1
    vb = gce * d1 - e * d21
    vc = gae * d2_ + e * d21
    va = det - (gce * d1 + gae * d2_)
    eps = 1e-12
    rcp = 1.0 / ((va + vb) + vc + eps)
    v = vb * rcp
    w = vc * rcp
    s1 = d4 - d3
    s2 = d5 - d6
    tbc = s1 / ((s1 + s2) + eps)
    m = (va <= 0) & (s1 >= 0) & (s2 >= 0)
    v = jnp.where(m, 1.0 - tbc, v)
    w = jnp.where(m, tbc, w)
    tac = d2_ / ((d2_ - d6) + eps)
    m = (vb <= 0) & (d2_ >= 0) & (d6 <= 0)
    v = jnp.where(m, 0.0, v)
    w = jnp.where(m, tac, w)
    m = (d6 >= 0) & (d5 <= d6)
    v = jnp.where(m, 0.0, v)
    w = jnp.where(m, 1.0, w)
    tab = d1 / ((d1 - d3) + eps)
    m = (vc <= 0) & (d1 >= 0) & (d3 <= 0)
    v = jnp.where(m, tab, v)
    w = jnp.where(m, 0.0, w)
    m = (d3 >= 0) & (d4 <= d3)
    v = jnp.where(m, 1.0, v)
    w = jnp.where(m, 0.0, w)
    m = (d1 <= 0) & (d2_ <= 0)
    v = jnp.where(m, 0.0, v)
    w = jnp.where(m, 0.0, w)

    clx = ftab_ref[0:1, :] + (v * abx + w * acx)
    cly = ftab_ref[1:2, :] + (v * aby + w * acy)
    clz = ftab_ref[2:3, :] + (v * abz + w * acz)
    ddx = px - clx
    ddy = py - cly
    ddz = pz - clz
    dsq = (ddx * ddx + ddy * ddy) + ddz * ddz

    ids_f = lax.broadcasted_iota(_i32, (_BQ, _F), 1)
    packed_a = jnp.bitwise_or(
        jnp.bitwise_and(lax.bitcast_convert_type(dsq, _i32), _i32(-2048)), ids_f
    )
    ma = jnp.min(packed_a, axis=1, keepdims=True)
    eqa = ids_f == jnp.bitwise_and(ma, _i32(2047))
    bcx = jnp.sum(jnp.where(eqa, clx, 0.0), axis=1, keepdims=True)
    bcy = jnp.sum(jnp.where(eqa, cly, 0.0), axis=1, keepdims=True)
    bcz = jnp.sum(jnp.where(eqa, clz, 0.0), axis=1, keepdims=True)

    dsx = px - bcx
    dsy = py - bcy
    dsz = pz - bcz
    t0 = dsx + 1e-10
    t1 = dsy + 1e-10
    t2 = dsz + 1e-10
    udf_s = jnp.sqrt((t0 * t0 + t1 * t1) + t2 * t2)

    dxt = px - tgtt_ref[0:1, :]
    dyt = py - tgtt_ref[1:2, :]
    dzt = pz - tgtt_ref[2:3, :]
    d2t = (dxt * dxt + dyt * dyt) + dzt * dzt

    ids_t = lax.broadcasted_iota(_i32, (_BQ, _T), 1)
    packed = jnp.bitwise_or(
        jnp.bitwise_and(lax.bitcast_convert_type(d2t, _i32), _i32(-4096)), ids_t
    )
    sl = [packed[:, j * _G : (j + 1) * _G] for j in range(_NO)]
    while len(sl) > 1:
        sl = [jnp.minimum(sl[2 * j], sl[2 * j + 1]) for j in range(len(sl) // 2)]
    gm = sl[0]

    odiv = lax.broadcasted_iota(_i32, (_BQ, 3 * _NO), 1) // 3
    ids_g = lax.broadcasted_iota(_i32, (_BQ, _G), 1)
    acc = jnp.zeros((_BQ, 3 * _NO), _f32)
    s = jnp.zeros((_BQ, 1), _f32)
    for k in range(_K):
        mk = jnp.min(gm, axis=1, keepdims=True)
        eq = ids_g == jnp.bitwise_and(mk, _i32(_G - 1))
        g1 = eq.astype(jnp.bfloat16)
        both = lax.dot_general(
            g1,
            tgq_ref[:, :],
            (((1,), (0,)), ((), ())),
            preferred_element_type=_f32,
        )
        tmpk = both[:, 0 : 3 * _NO] + both[:, 3 * _NO : 6 * _NO]
        dk = lax.bitcast_convert_type(jnp.bitwise_and(mk, _i32(-4096)), _f32)
        wk = 1.0 / (dk + 1e-8)
        s = s + wk
        ok = jnp.right_shift(jnp.bitwise_and(mk, _i32(4095)), _i32(_G.bit_length() - 1))
        acc = acc + jnp.where(odiv == ok, tmpk * wk, 0.0)
        if k < _K - 1:
            gm = jnp.where(eq, _IMAX, gm)

    def _csum(c):
        t = [acc[:, 3 * o + c : 3 * o + c + 1] for o in range(_NO)]
        while len(t) > 1:
            t = [t[2 * j] + t[2 * j + 1] for j in range(len(t) // 2)]
        return t[0]

    spx = _csum(0) / s
    spy = _csum(1) / s
    spz = _csum(2) / s
    dtx = px - spx
    dty = py - spy
    dtz = pz - spz
    r0 = dtx + 1e-10
    r1 = dty + 1e-10
    r2 = dtz + 1e-10
    udf_t = jnp.sqrt((r0 * r0 + r1 * r1) + r2 * r2)

    err = (
        (jnp.abs(dsx - dtx) + jnp.abs(dsy - dty)) + jnp.abs(dsz - dtz)
    ) + jnp.abs(udf_s - udf_t)
    tot = jnp.sum(err)

    ri = lax.broadcasted_iota(_i32, (8, 128), 0)
    ci = lax.broadcasted_iota(_i32, (8, 128), 1)
    upd = jnp.where((ri == 0) & (ci == 0), tot, 0.0)

    @pl.when(i == 0)
    def _():
        out_ref[:, :] = jnp.zeros((8, 128), _f32)

    out_ref[:, :] += upd


@jax.jit
def kernel(src_v, src_f, tgt_points, noise):
    idx_flat = src_f.T.reshape(-1)
    table = jnp.zeros((_V, _DPAD), _f32).at[:, 0:3].set(src_v)
    rows = _make_sc_gather()(table, idx_flat)
    ftab = pl.pallas_call(
        _prep_tab_kernel,
        out_shape=jax.ShapeDtypeStruct((24, _F), _f32),
    )(rows[:, 0:3].T)

    center = ftab[0:3, :] + (ftab[3:6, :] + ftab[6:9, :]) / 3.0
    qp = jnp.concatenate(
        [(tgt_points[:, None, :] + _STD * noise).reshape(-1, 3), center.T], axis=0
    )
    tgtt = jnp.zeros((8, _T), _f32).at[0:3, :].set(tgt_points.T)
    tgq = jnp.transpose(tgt_points.reshape(_NO, _G, 3), (1, 0, 2)).reshape(
        _G, 3 * _NO
    )
    tgq_hi = tgq.astype(jnp.bfloat16)
    tgq_lo = (tgq - tgq_hi.astype(_f32)).astype(jnp.bfloat16)
    tgq2 = jnp.concatenate([tgq_hi, tgq_lo], axis=1)

    acc = pl.pallas_call(
        _main_kernel,
        grid=(_Q // _BQ,),
        in_specs=[
            pl.BlockSpec((_BQ, 3), lambda i: (i, 0)),
            pl.BlockSpec((24, _F), lambda i: (0, 0)),
            pl.BlockSpec((8, _T), lambda i: (0, 0)),
            pl.BlockSpec((_G, 6 * _NO), lambda i: (0, 0)),
        ],
        out_specs=pl.BlockSpec((8, 128), lambda i: (0, 0)),
        out_shape=jax.ShapeDtypeStruct((8, 128), _f32),
    )(qp, ftab, tgtt, tgq2)

    return acc[0, 0] / _Q
